# Initial kernel scaffold; baseline (speedup 1.0000x reference)
import jax
import jax.numpy as jnp
from jax.experimental import pallas as pl


def kernel(tensor):
    raise NotImplementedError("write your pallas kernel here")


# speedup vs baseline: 1.0428x; 1.0428x over previous
import jax
import jax.numpy as jnp
from jax.experimental import pallas as pl
from jax.experimental.pallas import tpu as pltpu

_QMAX = 127.0
_NBLK = 16


def _body(x_ref, o_ref, acc_ref):
    i = pl.program_id(0)

    @pl.when(i == 0)
    def _init():
        acc_ref[0] = 0.0

    @pl.when(i < _NBLK)
    def _reduce():
        acc_ref[0] = jnp.maximum(acc_ref[0], jnp.max(jnp.abs(x_ref[...])))

    @pl.when(i >= _NBLK)
    def _quant():
        scale = acc_ref[0] / _QMAX
        q = jnp.round(jnp.clip(x_ref[...] / scale, -_QMAX, _QMAX))
        o_ref[...] = scale * q


def kernel(tensor):
    shape = tensor.shape
    x = tensor.reshape(-1, shape[-1])
    rows = x.shape[0]
    blk = rows // _NBLK

    out = pl.pallas_call(
        _body,
        grid=(2 * _NBLK,),
        in_specs=[
            pl.BlockSpec((blk, shape[-1]), lambda i: (i % _NBLK, 0)),
        ],
        out_specs=pl.BlockSpec(
            (blk, shape[-1]),
            lambda i: (jnp.where(i < _NBLK, 0, i - _NBLK), 0),
        ),
        out_shape=jax.ShapeDtypeStruct(x.shape, x.dtype),
        scratch_shapes=[pltpu.SMEM((1,), jnp.float32)],
    )(x)
    return out.reshape(shape)


# speedup vs baseline: 1.6894x; 1.6201x over previous
import jax
import jax.numpy as jnp
from jax.experimental import pallas as pl
from jax.experimental.pallas import tpu as pltpu

_QMAX = 127.0
_NCHUNK = 32


def _body(x_hbm, o_hbm, buf, sem_in, sem_out):
    rows = buf.shape[0]
    r = rows // _NCHUNK

    def _in_copy(c):
        return pltpu.make_async_copy(
            x_hbm.at[pl.ds(c * r, r)], buf.at[pl.ds(c * r, r)], sem_in)

    def _out_copy(c):
        return pltpu.make_async_copy(
            buf.at[pl.ds(c * r, r)], o_hbm.at[pl.ds(c * r, r)], sem_out)

    for c in range(_NCHUNK):
        _in_copy(c).start()

    m = jnp.float32(0.0)
    for c in range(_NCHUNK):
        _in_copy(c).wait()
        m = jnp.maximum(m, jnp.max(jnp.abs(buf[pl.ds(c * r, r)])))

    scale = m / _QMAX
    for c in range(_NCHUNK):
        x = buf[pl.ds(c * r, r)]
        q = jnp.round(jnp.clip(x / scale, -_QMAX, _QMAX))
        buf[pl.ds(c * r, r)] = scale * q
        _out_copy(c).start()

    for c in range(_NCHUNK):
        _out_copy(c).wait()


def kernel(tensor):
    shape = tensor.shape
    x = tensor.reshape(-1, shape[-1])

    out = pl.pallas_call(
        _body,
        in_specs=[pl.BlockSpec(memory_space=pl.ANY)],
        out_specs=pl.BlockSpec(memory_space=pl.ANY),
        out_shape=jax.ShapeDtypeStruct(x.shape, x.dtype),
        scratch_shapes=[
            pltpu.VMEM(x.shape, jnp.float32),
            pltpu.SemaphoreType.DMA,
            pltpu.SemaphoreType.DMA,
        ],
    )(x)
    return out.reshape(shape)
